# baseline (speedup 1.0000x reference)
import functools
import math

import jax
import jax.numpy as jnp
from jax import lax
from jax.experimental import pallas as pl
from jax.experimental.pallas import tpu as pltpu
from jax.experimental.pallas import tpu_sc as plsc

D = 128
SCALE = math.sqrt(float(D))

NC = 2
NS = 16
NW = NC * NS

IDXW = 128
N_BUF = 5
LEAD = N_BUF - 1


def _scale_body(t_ref, o_ref):
    o_ref[...] = t_ref[...] * SCALE


def _scale_table(table):
    v, d = table.shape
    blk = 2000
    grid = (v + blk - 1) // blk
    return pl.pallas_call(
        _scale_body,
        grid=(grid,),
        in_specs=[pl.BlockSpec((blk, d), lambda i: (i, 0))],
        out_specs=pl.BlockSpec((blk, d), lambda i: (i, 0)),
        out_shape=jax.ShapeDtypeStruct((v, d), table.dtype),
    )(table)


@functools.partial(jax.jit, static_argnames=("n_rows",))
def _sc_gather(idx2d, table, n_rows):
    rows_per_w = n_rows // NW
    n = rows_per_w
    mesh = plsc.VectorSubcoreMesh(
        core_axis_name="c", subcore_axis_name="s", num_cores=NC, num_subcores=NS
    )

    @functools.partial(
        pl.kernel,
        out_type=jax.ShapeDtypeStruct((n_rows * IDXW, D), jnp.float32),
        mesh=mesh,
        scratch_types=[
            pltpu.VMEM((rows_per_w, IDXW), jnp.int32),
            pltpu.VMEM((N_BUF, IDXW, D), jnp.float32),
            [pltpu.SemaphoreType.DMA] * N_BUF,
            [pltpu.SemaphoreType.DMA] * N_BUF,
        ],
    )
    def gather_kernel(idx_hbm, table_hbm, out_hbm, idx_v, rows_v, gsem, ssem):
        wid = lax.axis_index("s") * NC + lax.axis_index("c")
        row0 = wid * rows_per_w

        def fire_g(cc, b):
            pltpu.async_copy(table_hbm.at[idx_v.at[cc]], rows_v.at[b], gsem[b])

        def wait_g(cc, b):
            pltpu.make_async_copy(
                table_hbm.at[idx_v.at[cc]], rows_v.at[b], gsem[b]
            ).wait()

        def fire_s(cc, b):
            pltpu.async_copy(
                rows_v.at[b], out_hbm.at[pl.ds((row0 + cc) * IDXW, IDXW)], ssem[b]
            )

        def wait_s(b):
            pltpu.make_async_copy(
                rows_v.at[b], out_hbm.at[pl.ds(row0 * IDXW, IDXW)], ssem[b]
            ).wait()

        pltpu.sync_copy(idx_hbm.at[pl.ds(row0, rows_per_w)], idx_v)

        for b in range(LEAD):
            fire_g(b, b)
        wait_g(0, 0)
        fire_s(0, 0)
        fire_g(LEAD, LEAD)
        for cc in range(1, N_BUF):
            wait_g(cc, cc)
            fire_s(cc, cc)
            bb = (cc + LEAD) % N_BUF
            wait_s(bb)
            fire_g(cc + LEAD, bb)

        @pl.loop(N_BUF, n - N_BUF, step=N_BUF)
        def _group(g):
            for u in range(N_BUF):
                cc = g + u
                wait_g(cc, u)
                fire_s(cc, u)
                bb = (u + LEAD) % N_BUF
                wait_s(bb)
                fire_g(cc + LEAD, bb)

        wait_g(n - N_BUF, 0)
        fire_s(n - N_BUF, 0)
        wait_s(LEAD)
        fire_g(n - 1, LEAD)
        for cc in range(n - N_BUF + 1, n):
            b = cc % N_BUF
            wait_g(cc, b)
            fire_s(cc, b)
        for b in range(N_BUF):
            wait_s(b)

    return gather_kernel(idx2d, table)


def kernel(indices, table):
    b, t = indices.shape
    total = b * t
    n_rows = total // IDXW
    idx2d = indices.reshape(n_rows, IDXW)
    scaled = _scale_table(table)
    out = _sc_gather(idx2d, scaled, n_rows)
    return out.reshape(b, t, D)


# speedup vs baseline: 1.1494x; 1.1494x over previous
import functools
import math

import jax
import jax.numpy as jnp
from jax import lax
from jax.experimental import pallas as pl
from jax.experimental.pallas import tpu as pltpu
from jax.experimental.pallas import tpu_sc as plsc

D = 128
SCALE = math.sqrt(float(D))

NC = 2
NS = 16
NW = NC * NS

IDXW = 128
N_BUF = 5
LEAD = N_BUF - 1


def _scale_body(t_ref, o_ref):
    o_ref[...] = t_ref[...] * SCALE


def _scale_table(table):
    v, d = table.shape
    blk = 2000
    grid = (v + blk - 1) // blk
    return pl.pallas_call(
        _scale_body,
        grid=(grid,),
        in_specs=[pl.BlockSpec((blk, d), lambda i: (i, 0))],
        out_specs=pl.BlockSpec((blk, d), lambda i: (i, 0)),
        out_shape=jax.ShapeDtypeStruct((v, d), table.dtype),
    )(table)


@functools.partial(jax.jit, static_argnames=("n_rows",))
def _sc_gather(idx2d, table, n_rows):
    rows_per_w = n_rows // NW
    n = rows_per_w
    mesh = plsc.VectorSubcoreMesh(
        core_axis_name="c", subcore_axis_name="s", num_cores=NC, num_subcores=NS
    )

    @functools.partial(
        pl.kernel,
        out_type=jax.ShapeDtypeStruct((n_rows * IDXW, D), jnp.float32),
        mesh=mesh,
        scratch_types=[
            pltpu.VMEM((rows_per_w, IDXW), jnp.int32),
            pltpu.VMEM((N_BUF, IDXW, D), jnp.float32),
            [pltpu.SemaphoreType.DMA] * N_BUF,
            [pltpu.SemaphoreType.DMA] * N_BUF,
        ],
    )
    def gather_kernel(idx_hbm, table_hbm, out_hbm, idx_v, rows_v, gsem, ssem):
        wid = lax.axis_index("s") * NC + lax.axis_index("c")
        row0 = wid * rows_per_w

        def fire_g(cc, b):
            pltpu.async_copy(table_hbm.at[idx_v.at[cc]], rows_v.at[b], gsem[b])

        def wait_g(cc, b):
            pltpu.make_async_copy(
                table_hbm.at[idx_v.at[cc]], rows_v.at[b], gsem[b]
            ).wait()

        def fire_s(cc, b):
            pltpu.async_copy(
                rows_v.at[b], out_hbm.at[pl.ds((row0 + cc) * IDXW, IDXW)], ssem[b]
            )

        def wait_s(b):
            pltpu.make_async_copy(
                rows_v.at[b], out_hbm.at[pl.ds(row0 * IDXW, IDXW)], ssem[b]
            ).wait()

        def scale_buf(b):
            rv = rows_v.at[b]

            @pl.loop(0, IDXW // 4)
            def _rows(r4):
                r = r4 * 4
                for dr in range(4):
                    for q in range(D // 16):
                        sl = pl.ds(q * 16, 16)
                        rv[r + dr, sl] = rv[r + dr, sl] * SCALE

        pltpu.sync_copy(idx_hbm.at[pl.ds(row0, rows_per_w)], idx_v)

        for b in range(LEAD):
            fire_g(b, b)
        wait_g(0, 0)
        scale_buf(0)
        fire_s(0, 0)
        fire_g(LEAD, LEAD)
        for cc in range(1, N_BUF):
            wait_g(cc, cc)
            scale_buf(cc)
            fire_s(cc, cc)
            bb = (cc + LEAD) % N_BUF
            wait_s(bb)
            fire_g(cc + LEAD, bb)

        @pl.loop(N_BUF, n - N_BUF, step=N_BUF)
        def _group(g):
            for u in range(N_BUF):
                cc = g + u
                wait_g(cc, u)
                scale_buf(u)
                fire_s(cc, u)
                bb = (u + LEAD) % N_BUF
                wait_s(bb)
                fire_g(cc + LEAD, bb)

        wait_g(n - N_BUF, 0)
        scale_buf(0)
        fire_s(n - N_BUF, 0)
        wait_s(LEAD)
        fire_g(n - 1, LEAD)
        for cc in range(n - N_BUF + 1, n):
            b = cc % N_BUF
            wait_g(cc, b)
            scale_buf(b)
            fire_s(cc, b)
        for b in range(N_BUF):
            wait_s(b)

    return gather_kernel(idx2d, table)


def kernel(indices, table):
    b, t = indices.shape
    total = b * t
    n_rows = total // IDXW
    idx2d = indices.reshape(n_rows, IDXW)
    out = _sc_gather(idx2d, table, n_rows)
    return out.reshape(b, t, D)
